# baseline (speedup 1.0000x reference)
import functools

import jax
import jax.numpy as jnp
from jax import lax
from jax.experimental import pallas as pl
from jax.experimental.pallas import tpu as pltpu
from jax.experimental.pallas import tpu_sc as plsc

N = 10000
NP = 10240
D_IN = 1024
H = 4
C = 256
F = 128
NCH = 8
E_RAW = 160000
EN = E_RAW + N
EP = 172032
NC = 2
NS = 16
EA = EP // (NC * NS)
BA = EA // 128
EC = EP // NS
BLK = 32
GB = 24
GE = GB * BLK
NG = EC // GE
RPT = NP // NS
TILE_N = 400

_mesh = functools.partial(
    plsc.VectorSubcoreMesh, core_axis_name="c", subcore_axis_name="s",
    num_cores=NC, num_subcores=NS)


def _iota16():
    return lax.iota(jnp.int32, 16)


def _splat(ref1d, idx):
    return plsc.load_gather(ref1d, [jnp.full((16,), idx, jnp.int32)])


def _mm_body(x_ref, w_ref, ms_ref, md_ref, hch_ref, as_ref, ad_ref):
    hb = jnp.dot(x_ref[...], w_ref[...],
                 preferred_element_type=jnp.float32,
                 precision=lax.Precision.HIGHEST)
    for cg in range(NCH):
        hch_ref[cg] = hb[:, cg * F:(cg + 1) * F].astype(jnp.bfloat16)
    as_ref[...] = jnp.dot(hb, ms_ref[...], preferred_element_type=jnp.float32,
                          precision=lax.Precision.HIGHEST)
    ad_ref[...] = jnp.dot(hb, md_ref[...], preferred_element_type=jnp.float32,
                          precision=lax.Precision.HIGHEST)


def _tc_matmul(x, W, Ms, Md):
    return pl.pallas_call(
        _mm_body,
        grid=(N // TILE_N,),
        in_specs=[
            pl.BlockSpec((TILE_N, D_IN), lambda i: (i, 0)),
            pl.BlockSpec((D_IN, H * C), lambda i: (0, 0)),
            pl.BlockSpec((H * C, H), lambda i: (0, 0)),
            pl.BlockSpec((H * C, H), lambda i: (0, 0)),
        ],
        out_specs=[
            pl.BlockSpec((NCH, TILE_N, F), lambda i: (0, i, 0)),
            pl.BlockSpec((TILE_N, H), lambda i: (i, 0)),
            pl.BlockSpec((TILE_N, H), lambda i: (i, 0)),
        ],
        out_shape=[
            jax.ShapeDtypeStruct((NCH, N, F), jnp.bfloat16),
            jax.ShapeDtypeStruct((N, H), jnp.float32),
            jax.ShapeDtypeStruct((N, H), jnp.float32),
        ],
    )(x, W, Ms, Md)


def _edge_body(asrc_h, adst_h, srcp_h, dstp_h,
               exh_h, dpart_h,
               asv, adv, sv, dv, exbuf, zbuf,
               dsp0, dsp1, dsp2, dsp3, sem):
    c = lax.axis_index("c")
    s = lax.axis_index("s")
    wid = s * NC + c
    dsps = [dsp0, dsp1, dsp2, dsp3]

    def _z(i, carry):
        zbuf[pl.ds(i * 16, 16)] = jnp.zeros((16,), jnp.float32)
        return carry
    lax.fori_loop(0, RPT // 16, _z, 0)
    for hh in range(H):
        pltpu.sync_copy(zbuf, dsps[hh].at[pl.ds(s * RPT, RPT)])

    pltpu.sync_copy(asrc_h, asv)
    pltpu.sync_copy(adst_h, adv)
    plsc.subcore_barrier()

    base = wid * EA

    def _blk(b, carry):
        eb = base + b * 128
        pltpu.sync_copy(srcp_h.at[pl.ds(eb, 128)], sv)
        pltpu.sync_copy(dstp_h.at[pl.ds(eb, 128)], dv)
        for g in range(8):
            s16 = sv[pl.ds(g * 16, 16)] * H
            d16 = dv[pl.ds(g * 16, 16)] * H
            eid = eb + g * 16 + _iota16()
            valid = eid < EN
            for hh in range(H):
                av = plsc.load_gather(asv, [s16 + hh])
                bv = plsc.load_gather(adv, [d16 + hh])
                al = av + bv
                al = jnp.maximum(al, 0.2 * al)
                ev = jnp.where(valid, jnp.exp(al), 0.0)
                exbuf[hh, pl.ds(b * 128 + g * 16, 16)] = ev
        for hh in range(H):
            pltpu.sync_copy(exbuf.at[hh, pl.ds(b * 128, 128)],
                            dsps[hh].at[dv], add=True)
        return carry
    lax.fori_loop(0, BA, _blk, 0)

    for hh in range(H):
        pltpu.sync_copy(exbuf.at[hh], exh_h.at[hh, pl.ds(base, EA)])
    plsc.subcore_barrier()
    for hh in range(H):
        pltpu.sync_copy(dsps[hh].at[pl.ds(s * RPT, RPT)],
                        dpart_h.at[c, hh, pl.ds(s * RPT, RPT)])


def _sc_edges(a_src_flat, a_dst_flat, srcp, dstp):
    return pl.kernel(
        _edge_body,
        out_type=[
            jax.ShapeDtypeStruct((H, EP), jnp.float32),
            jax.ShapeDtypeStruct((NC, H, NP), jnp.float32),
        ],
        mesh=_mesh(),
        compiler_params=pltpu.CompilerParams(needs_layout_passes=False),
        scratch_types=[
            pltpu.VMEM((N * H,), jnp.float32),
            pltpu.VMEM((N * H,), jnp.float32),
            pltpu.VMEM((128,), jnp.int32),
            pltpu.VMEM((128,), jnp.int32),
            pltpu.VMEM((H, EA), jnp.float32),
            pltpu.VMEM((RPT,), jnp.float32),
            pltpu.VMEM_SHARED((NP,), jnp.float32),
            pltpu.VMEM_SHARED((NP,), jnp.float32),
            pltpu.VMEM_SHARED((NP,), jnp.float32),
            pltpu.VMEM_SHARED((NP,), jnp.float32),
            pltpu.SemaphoreType.DMA,
        ],
    )(a_src_flat, a_dst_flat, srcp, dstp)


def _agg_body(hflat_h, srcp_h, dst2_h, exh_h, dpart_h, bias_h,
              outp_h,
              sv_big, dv2, iv2, ex_big,
              rbf0, rbf1, rbf2, rbf3, rows0, rows1, rows2, rows3,
              dtmp, rinvf, bv, accsp,
              sem_g0, sem_g1, sem_g2, sem_g3,
              sem_s0, sem_s1, sem_s2, sem_s3):
    c = lax.axis_index("c")
    s = lax.axis_index("s")
    row0 = s * RPT
    rbf_b = (rbf0, rbf1, rbf2, rbf3)
    rows_b = (rows0, rows1, rows2, rows3)
    sem_g = (sem_g0, sem_g1, sem_g2, sem_g3)
    sem_s = (sem_s0, sem_s1, sem_s2, sem_s3)

    for hh in range(H):
        pltpu.sync_copy(dpart_h.at[0, hh, pl.ds(row0, RPT)],
                        rinvf.at[pl.ds(hh * RPT, RPT)])
        pltpu.sync_copy(dpart_h.at[1, hh, pl.ds(row0, RPT)],
                        dtmp.at[pl.ds(hh * RPT, RPT)])

    def _r(i, carry):
        a = rinvf[pl.ds(i * 16, 16)]
        b = dtmp[pl.ds(i * 16, 16)]
        rinvf[pl.ds(i * 16, 16)] = 1.0 / (a + b + 1e-16)
        return carry
    lax.fori_loop(0, RPT * H // 16, _r, 0)

    def _chunk(ch, carry0):
        cg = c * (NCH // NC) + ch
        head = cg >> 1
        fbase = cg * F

        def _z(r, carry):
            for k in range(8):
                rows0[r, pl.ds(k * 16, 16)] = jnp.zeros((16,), jnp.float32)
            return carry
        lax.fori_loop(0, BLK, _z, 0)

        def _zc(k, carry):
            pltpu.sync_copy(rows0, accsp.at[pl.ds(row0 + k * BLK, BLK)])
            return carry
        lax.fori_loop(0, RPT // BLK, _zc, 0)
        plsc.subcore_barrier()

        def _group(gi, carry):
            geb = s * EC + gi * GE
            gbb = s * (EC // BLK) + gi * GB
            pltpu.sync_copy(srcp_h.at[pl.ds(geb, GE)], sv_big)
            pltpu.sync_copy(dst2_h.at[pl.ds(gbb, GB)], dv2)
            pltpu.sync_copy(exh_h.at[head, pl.ds(geb, GE)], ex_big)
            for kb in range(GB):
                for j in range(BLK // 16):
                    iv2[kb, pl.ds(j * 16, 16)] = (
                        sv_big[pl.ds(kb * BLK + j * 16, 16)] + cg * N)

            def _gat(kb, buf):
                return pltpu.async_copy(hflat_h.at[iv2.at[kb]],
                                        rbf_b[buf], sem_g[buf])

            _gat(0, 0)

            def _quad(sb, carry2):
                for q in range(4):
                    kb = sb * 4 + q
                    buf = q
                    nbuf = (q + 1) % 4
                    pltpu.make_async_copy(hflat_h.at[iv2.at[kb]],
                                          rbf_b[buf], sem_g[buf]).wait()

                    @pl.when(kb >= 4)
                    def _():
                        pltpu.make_async_copy(rows_b[buf],
                                              accsp.at[dv2.at[kb - 4]],
                                              sem_s[buf]).wait()

                    @plsc.parallel_loop(0, BLK, step=1, unroll=4)
                    def _scale(j):
                        sc = _splat(ex_big, kb * BLK + j)
                        for k in range(4):
                            w = rbf_b[buf][j, pl.ds(k * 16, 16)]
                            ev = plsc.bitcast(jnp.left_shift(w, 16),
                                              jnp.float32)
                            od = plsc.bitcast(
                                jnp.bitwise_and(w, jnp.int32(-65536)),
                                jnp.float32)
                            rows_b[buf][j, pl.ds(k * 32, 16)] = ev * sc
                            rows_b[buf][j, pl.ds(k * 32 + 16, 16)] = od * sc

                    pltpu.async_copy(rows_b[buf], accsp.at[dv2.at[kb]],
                                     sem_s[buf], add=True)

                    @pl.when(kb + 1 < GB)
                    def _():
                        _gat(kb + 1, nbuf)
                return carry2
            lax.fori_loop(0, GB // 4, _quad, 0)
            for kb in range(GB - 4, GB):
                pltpu.make_async_copy(rows_b[kb % 4], accsp.at[dv2.at[kb]],
                                      sem_s[kb % 4]).wait()
            return carry
        lax.fori_loop(0, NG, _group, 0)
        plsc.subcore_barrier()

        pltpu.sync_copy(bias_h.at[pl.ds(fbase, F)], bv)

        def _fink(k5, carry):
            r0 = row0 + k5 * BLK
            pltpu.sync_copy(accsp.at[pl.ds(r0, BLK)], rows0)

            @plsc.parallel_loop(0, BLK, step=1, unroll=4)
            def _fin(r):
                sc = _splat(rinvf, head * RPT + k5 * BLK + r)
                for k in range(8):
                    rows0[r, pl.ds(k * 16, 16)] = jnp.maximum(
                        rows0[r, pl.ds(k * 16, 16)] * sc
                        + bv[pl.ds(k * 16, 16)], 0.0)
            pltpu.sync_copy(rows0, outp_h.at[pl.ds(r0, BLK), pl.ds(fbase, F)])
            return carry
        lax.fori_loop(0, RPT // BLK, _fink, 0)
        plsc.subcore_barrier()
        return carry0
    lax.fori_loop(0, NCH // NC, _chunk, 0)


def _sc_aggregate(hflat, srcp, dst2, exh, dpart, bias):
    return pl.kernel(
        _agg_body,
        out_type=jax.ShapeDtypeStruct((NP, H * C), jnp.float32),
        mesh=_mesh(),
        compiler_params=pltpu.CompilerParams(needs_layout_passes=False,
                                             use_tc_tiling_on_sc=False),
        scratch_types=[
            pltpu.VMEM((GE,), jnp.int32),
            pltpu.VMEM((GB, BLK), jnp.int32),
            pltpu.VMEM((GB, BLK), jnp.int32),
            pltpu.VMEM((GE,), jnp.float32),
            pltpu.VMEM((BLK, F // 2), jnp.int32),
            pltpu.VMEM((BLK, F // 2), jnp.int32),
            pltpu.VMEM((BLK, F // 2), jnp.int32),
            pltpu.VMEM((BLK, F // 2), jnp.int32),
            pltpu.VMEM((BLK, F), jnp.float32),
            pltpu.VMEM((BLK, F), jnp.float32),
            pltpu.VMEM((BLK, F), jnp.float32),
            pltpu.VMEM((BLK, F), jnp.float32),
            pltpu.VMEM((RPT * H,), jnp.float32),
            pltpu.VMEM((RPT * H,), jnp.float32),
            pltpu.VMEM((F,), jnp.float32),
            pltpu.VMEM_SHARED((NP, F), jnp.float32),
            pltpu.SemaphoreType.DMA,
            pltpu.SemaphoreType.DMA,
            pltpu.SemaphoreType.DMA,
            pltpu.SemaphoreType.DMA,
            pltpu.SemaphoreType.DMA,
            pltpu.SemaphoreType.DMA,
            pltpu.SemaphoreType.DMA,
            pltpu.SemaphoreType.DMA,
        ],
    )(hflat, srcp, dst2, exh, dpart, bias)


def kernel(x, edge_index, W, att_src, att_dst, bias):
    eye = jnp.eye(H, dtype=jnp.float32)
    Ms = (att_src.reshape(H, C)[:, :, None] * eye[:, None, :]).reshape(H * C, H)
    Md = (att_dst.reshape(H, C)[:, :, None] * eye[:, None, :]).reshape(H * C, H)

    loop = jnp.arange(N, dtype=edge_index.dtype)
    pad = jnp.arange(EP - EN, dtype=edge_index.dtype) % N
    srcp = jnp.concatenate([edge_index[0], loop, pad]).astype(jnp.int32)
    dstp = jnp.concatenate([edge_index[1], loop, pad]).astype(jnp.int32)

    hch, a_src, a_dst = _tc_matmul(x, W, Ms, Md)
    hflat = (hch.reshape(NCH, N, F // 32, 2, 16)
             .transpose(0, 1, 2, 4, 3).reshape(NCH * N, F // 2, 2))
    hflat = jax.lax.bitcast_convert_type(hflat, jnp.int32)

    exh, dpart = _sc_edges(a_src.reshape(N * H), a_dst.reshape(N * H),
                           srcp, dstp)
    outp = _sc_aggregate(hflat, srcp, dstp.reshape(EP // BLK, BLK),
                         exh, dpart, bias)
    return outp[:N]


# speedup vs baseline: 1.5002x; 1.5002x over previous
import functools

import jax
import jax.numpy as jnp
from jax import lax
from jax.experimental import pallas as pl
from jax.experimental.pallas import tpu as pltpu
from jax.experimental.pallas import tpu_sc as plsc

N = 10000
NP = 10240
D_IN = 1024
H = 4
C = 256
F = 128
NCH = 8
E_RAW = 160000
EN = E_RAW + N
EP = 172032
NC = 2
NS = 16
EA = EP // (NC * NS)
BA = EA // 128
EC = EP // NS
BLK = 64
GB = 24
GE = GB * BLK
NG = EC // GE
RPT = NP // NS
TILE_N = 400

_mesh = functools.partial(
    plsc.VectorSubcoreMesh, core_axis_name="c", subcore_axis_name="s",
    num_cores=NC, num_subcores=NS)


def _iota16():
    return lax.iota(jnp.int32, 16)


def _splat(ref1d, idx):
    return plsc.load_gather(ref1d, [jnp.full((16,), idx, jnp.int32)])


def _mm_body(x_ref, w_ref, ms_ref, md_ref, hch_ref, as_ref, ad_ref):
    hb = jnp.dot(x_ref[...], w_ref[...], preferred_element_type=jnp.float32)
    for cg in range(NCH):
        hch_ref[cg] = hb[:, cg * F:(cg + 1) * F]
    as_ref[...] = jnp.dot(hb, ms_ref[...], preferred_element_type=jnp.float32)
    ad_ref[...] = jnp.dot(hb, md_ref[...], preferred_element_type=jnp.float32)


def _tc_matmul(x, W, Ms, Md):
    return pl.pallas_call(
        _mm_body,
        grid=(N // TILE_N,),
        in_specs=[
            pl.BlockSpec((TILE_N, D_IN), lambda i: (i, 0)),
            pl.BlockSpec((D_IN, H * C), lambda i: (0, 0)),
            pl.BlockSpec((H * C, H), lambda i: (0, 0)),
            pl.BlockSpec((H * C, H), lambda i: (0, 0)),
        ],
        out_specs=[
            pl.BlockSpec((NCH, TILE_N, F), lambda i: (0, i, 0)),
            pl.BlockSpec((TILE_N, H), lambda i: (i, 0)),
            pl.BlockSpec((TILE_N, H), lambda i: (i, 0)),
        ],
        out_shape=[
            jax.ShapeDtypeStruct((NCH, N, F), jnp.float32),
            jax.ShapeDtypeStruct((N, H), jnp.float32),
            jax.ShapeDtypeStruct((N, H), jnp.float32),
        ],
    )(x, W, Ms, Md)


def _edge_body(asrc_h, adst_h, srcp_h, dst2_h,
               exh_h, dpart_h,
               asv, adv, srcb, dstb2, exbuf, zbuf,
               dsp0, dsp1, dsp2, dsp3, sem):
    c = lax.axis_index("c")
    s = lax.axis_index("s")
    wid = s * NC + c
    dsps = [dsp0, dsp1, dsp2, dsp3]

    def _z(i, carry):
        zbuf[pl.ds(i * 16, 16)] = jnp.zeros((16,), jnp.float32)
        return carry
    lax.fori_loop(0, RPT // 16, _z, 0)
    for hh in range(H):
        pltpu.sync_copy(zbuf, dsps[hh].at[pl.ds(s * RPT, RPT)])

    pltpu.sync_copy(asrc_h, asv)
    pltpu.sync_copy(adst_h, adv)
    base = wid * EA
    pltpu.sync_copy(srcp_h.at[pl.ds(base, EA)], srcb)
    pltpu.sync_copy(dst2_h.at[wid], dstb2)
    plsc.subcore_barrier()

    def _blk(b, carry):
        eb = base + b * 128
        for g in range(8):
            s16 = srcb[pl.ds(b * 128 + g * 16, 16)] * H
            d16 = dstb2[b, pl.ds(g * 16, 16)] * H
            eid = eb + g * 16 + _iota16()
            valid = eid < EN
            for hh in range(H):
                av = plsc.load_gather(asv, [s16 + hh])
                bv = plsc.load_gather(adv, [d16 + hh])
                al = av + bv
                al = jnp.maximum(al, 0.2 * al)
                ev = jnp.where(valid, jnp.exp(al), 0.0)
                exbuf[hh, pl.ds(b * 128 + g * 16, 16)] = ev
        for hh in range(H):
            pltpu.sync_copy(exbuf.at[hh, pl.ds(b * 128, 128)],
                            dsps[hh].at[dstb2.at[b]], add=True)
        return carry
    lax.fori_loop(0, BA, _blk, 0)

    for hh in range(H):
        pltpu.sync_copy(exbuf.at[hh], exh_h.at[hh, pl.ds(base, EA)])
    plsc.subcore_barrier()
    for hh in range(H):
        pltpu.sync_copy(dsps[hh].at[pl.ds(s * RPT, RPT)],
                        dpart_h.at[c, hh, pl.ds(s * RPT, RPT)])


def _sc_edges(a_src_flat, a_dst_flat, srcp, dst2):
    return pl.kernel(
        _edge_body,
        out_type=[
            jax.ShapeDtypeStruct((H, EP), jnp.float32),
            jax.ShapeDtypeStruct((NC, H, NP), jnp.float32),
        ],
        mesh=_mesh(),
        compiler_params=pltpu.CompilerParams(needs_layout_passes=False),
        scratch_types=[
            pltpu.VMEM((N * H,), jnp.float32),
            pltpu.VMEM((N * H,), jnp.float32),
            pltpu.VMEM((EA,), jnp.int32),
            pltpu.VMEM((BA, 128), jnp.int32),
            pltpu.VMEM((H, EA), jnp.float32),
            pltpu.VMEM((RPT,), jnp.float32),
            pltpu.VMEM_SHARED((NP,), jnp.float32),
            pltpu.VMEM_SHARED((NP,), jnp.float32),
            pltpu.VMEM_SHARED((NP,), jnp.float32),
            pltpu.VMEM_SHARED((NP,), jnp.float32),
            pltpu.SemaphoreType.DMA,
        ],
    )(a_src_flat, a_dst_flat, srcp, dst2)


def _agg_body(hflat_h, srcp_h, dst2_h, exh_h, dpart_h, bias_h,
              outp_h,
              sv_big, dv2, iv2, ex_big,
              rows0, rows1, rows2, rows3,
              dtmp, rinvf, bv, accsp,
              sem_g0, sem_g1, sem_g2, sem_g3,
              sem_s0, sem_s1, sem_s2, sem_s3):
    c = lax.axis_index("c")
    s = lax.axis_index("s")
    row0 = s * RPT
    rows_b = (rows0, rows1, rows2, rows3)
    sem_g = (sem_g0, sem_g1, sem_g2, sem_g3)
    sem_s = (sem_s0, sem_s1, sem_s2, sem_s3)

    for hh in range(H):
        pltpu.sync_copy(dpart_h.at[0, hh, pl.ds(row0, RPT)],
                        rinvf.at[pl.ds(hh * RPT, RPT)])
        pltpu.sync_copy(dpart_h.at[1, hh, pl.ds(row0, RPT)],
                        dtmp.at[pl.ds(hh * RPT, RPT)])

    def _r(i, carry):
        a = rinvf[pl.ds(i * 16, 16)]
        b = dtmp[pl.ds(i * 16, 16)]
        rinvf[pl.ds(i * 16, 16)] = 1.0 / (a + b + 1e-16)
        return carry
    lax.fori_loop(0, RPT * H // 16, _r, 0)

    def _chunk(ch, carry0):
        cg = c * (NCH // NC) + ch
        head = cg >> 1
        fbase = cg * F

        def _z(r, carry):
            for k in range(8):
                rows0[r, pl.ds(k * 16, 16)] = jnp.zeros((16,), jnp.float32)
            return carry
        lax.fori_loop(0, BLK, _z, 0)

        def _zc(k, carry):
            pltpu.sync_copy(rows0, accsp.at[pl.ds(row0 + k * BLK, BLK)])
            return carry
        lax.fori_loop(0, RPT // BLK, _zc, 0)
        plsc.subcore_barrier()

        def _group(gi, carry):
            geb = s * EC + gi * GE
            gbb = s * (EC // BLK) + gi * GB
            pltpu.sync_copy(srcp_h.at[pl.ds(geb, GE)], sv_big)
            pltpu.sync_copy(dst2_h.at[pl.ds(gbb, GB)], dv2)
            pltpu.sync_copy(exh_h.at[head, pl.ds(geb, GE)], ex_big)
            for kb in range(GB):
                for j in range(BLK // 16):
                    iv2[kb, pl.ds(j * 16, 16)] = (
                        sv_big[pl.ds(kb * BLK + j * 16, 16)] + cg * N)

            def _gat(kb, buf):
                return pltpu.async_copy(hflat_h.at[iv2.at[kb]],
                                        rows_b[buf], sem_g[buf])

            _gat(0, 0)

            def _quad(sb, carry2):
                for q in range(4):
                    kb = sb * 4 + q
                    buf = q
                    nbuf = (q + 1) % 4
                    pltpu.make_async_copy(hflat_h.at[iv2.at[kb]],
                                          rows_b[buf], sem_g[buf]).wait()

                    @plsc.parallel_loop(0, BLK, step=1, unroll=4)
                    def _scale(j):
                        sc = _splat(ex_big, kb * BLK + j)
                        for k in range(8):
                            rows_b[buf][j, pl.ds(k * 16, 16)] = (
                                rows_b[buf][j, pl.ds(k * 16, 16)] * sc)

                    pltpu.async_copy(rows_b[buf], accsp.at[dv2.at[kb]],
                                     sem_s[buf], add=True)

                    @pl.when(kb >= 3)
                    def _():
                        pltpu.make_async_copy(rows_b[nbuf],
                                              accsp.at[dv2.at[kb - 3]],
                                              sem_s[nbuf]).wait()

                    @pl.when(kb + 1 < GB)
                    def _():
                        _gat(kb + 1, nbuf)
                return carry2
            lax.fori_loop(0, GB // 4, _quad, 0)
            for kb in range(GB - 3, GB):
                pltpu.make_async_copy(rows_b[kb % 4], accsp.at[dv2.at[kb]],
                                      sem_s[kb % 4]).wait()
            return carry
        lax.fori_loop(0, NG, _group, 0)
        plsc.subcore_barrier()

        pltpu.sync_copy(bias_h.at[pl.ds(fbase, F)], bv)

        def _fink(k5, carry):
            r0 = row0 + k5 * BLK
            pltpu.sync_copy(accsp.at[pl.ds(r0, BLK)], rows0)

            @plsc.parallel_loop(0, BLK, step=1, unroll=4)
            def _fin(r):
                sc = _splat(rinvf, head * RPT + k5 * BLK + r)
                for k in range(8):
                    rows0[r, pl.ds(k * 16, 16)] = jnp.maximum(
                        rows0[r, pl.ds(k * 16, 16)] * sc
                        + bv[pl.ds(k * 16, 16)], 0.0)
            pltpu.sync_copy(rows0, outp_h.at[pl.ds(r0, BLK), pl.ds(fbase, F)])
            return carry
        lax.fori_loop(0, RPT // BLK, _fink, 0)
        plsc.subcore_barrier()
        return carry0
    lax.fori_loop(0, NCH // NC, _chunk, 0)


def _sc_aggregate(hflat, srcp, dst2, exh, dpart, bias):
    return pl.kernel(
        _agg_body,
        out_type=jax.ShapeDtypeStruct((NP, H * C), jnp.float32),
        mesh=_mesh(),
        compiler_params=pltpu.CompilerParams(needs_layout_passes=False),
        scratch_types=[
            pltpu.VMEM((GE,), jnp.int32),
            pltpu.VMEM((GB, BLK), jnp.int32),
            pltpu.VMEM((GB, BLK), jnp.int32),
            pltpu.VMEM((GE,), jnp.float32),
            pltpu.VMEM((BLK, F), jnp.float32),
            pltpu.VMEM((BLK, F), jnp.float32),
            pltpu.VMEM((BLK, F), jnp.float32),
            pltpu.VMEM((BLK, F), jnp.float32),
            pltpu.VMEM((RPT * H,), jnp.float32),
            pltpu.VMEM((RPT * H,), jnp.float32),
            pltpu.VMEM((F,), jnp.float32),
            pltpu.VMEM_SHARED((NP, F), jnp.float32),
            pltpu.SemaphoreType.DMA,
            pltpu.SemaphoreType.DMA,
            pltpu.SemaphoreType.DMA,
            pltpu.SemaphoreType.DMA,
            pltpu.SemaphoreType.DMA,
            pltpu.SemaphoreType.DMA,
            pltpu.SemaphoreType.DMA,
            pltpu.SemaphoreType.DMA,
        ],
    )(hflat, srcp, dst2, exh, dpart, bias)


def kernel(x, edge_index, W, att_src, att_dst, bias):
    eye = jnp.eye(H, dtype=jnp.float32)
    Ms = (att_src.reshape(H, C)[:, :, None] * eye[:, None, :]).reshape(H * C, H)
    Md = (att_dst.reshape(H, C)[:, :, None] * eye[:, None, :]).reshape(H * C, H)

    loop = jnp.arange(N, dtype=edge_index.dtype)
    pad = jnp.arange(EP - EN, dtype=edge_index.dtype) % N
    srcp = jnp.concatenate([edge_index[0], loop, pad]).astype(jnp.int32)
    dstp = jnp.concatenate([edge_index[1], loop, pad]).astype(jnp.int32)

    hch, a_src, a_dst = _tc_matmul(x, W, Ms, Md)
    hflat = hch.reshape(NCH * N, F)

    exh, dpart = _sc_edges(a_src.reshape(N * H), a_dst.reshape(N * H),
                           srcp, dstp.reshape(NC * NS, BA, 128))
    outp = _sc_aggregate(hflat, srcp, dstp.reshape(EP // BLK, BLK),
                         exh, dpart, bias)
    return outp[:N]


# speedup vs baseline: 2.0698x; 1.3797x over previous
import functools

import jax
import jax.numpy as jnp
from jax import lax
from jax.experimental import pallas as pl
from jax.experimental.pallas import tpu as pltpu
from jax.experimental.pallas import tpu_sc as plsc

N = 10000
NP = 10240
D_IN = 1024
H = 4
C = 256
F = 128
NCH = 8
E_RAW = 160000
EN = E_RAW + N
EP = 172032
NC = 2
NS = 16
EA = EP // (NC * NS)
BA = EA // 128
EC = EP // NS
BLK = 64
GB = 24
GE = GB * BLK
NG = EC // GE
RPT = NP // NS
TILE_N = 400

_mesh = functools.partial(
    plsc.VectorSubcoreMesh, core_axis_name="c", subcore_axis_name="s",
    num_cores=NC, num_subcores=NS)


def _iota16():
    return lax.iota(jnp.int32, 16)


def _splat(ref1d, idx):
    return plsc.load_gather(ref1d, [jnp.full((16,), idx, jnp.int32)])


def _mm_body(x_ref, w_ref, ms_ref, md_ref, hch_ref, as_ref, ad_ref):
    hb = jnp.dot(x_ref[...], w_ref[...], preferred_element_type=jnp.float32)
    for cg in range(NCH):
        hch_ref[cg] = hb[:, cg * F:(cg + 1) * F].astype(jnp.bfloat16)
    as_ref[...] = jnp.dot(hb, ms_ref[...], preferred_element_type=jnp.float32)
    ad_ref[...] = jnp.dot(hb, md_ref[...], preferred_element_type=jnp.float32)


def _tc_matmul(x, W, Ms, Md):
    return pl.pallas_call(
        _mm_body,
        grid=(N // TILE_N,),
        in_specs=[
            pl.BlockSpec((TILE_N, D_IN), lambda i: (i, 0)),
            pl.BlockSpec((D_IN, H * C), lambda i: (0, 0)),
            pl.BlockSpec((H * C, H), lambda i: (0, 0)),
            pl.BlockSpec((H * C, H), lambda i: (0, 0)),
        ],
        out_specs=[
            pl.BlockSpec((NCH, TILE_N, F), lambda i: (0, i, 0)),
            pl.BlockSpec((TILE_N, H), lambda i: (i, 0)),
            pl.BlockSpec((TILE_N, H), lambda i: (i, 0)),
        ],
        out_shape=[
            jax.ShapeDtypeStruct((NCH, N, F), jnp.bfloat16),
            jax.ShapeDtypeStruct((N, H), jnp.float32),
            jax.ShapeDtypeStruct((N, H), jnp.float32),
        ],
    )(x, W, Ms, Md)


def _edge_body(asrc_h, adst_h, srcp_h, dst2_h,
               exh_h, dpart_h,
               asv, adv, srcb, dstb2, exbuf, zbuf,
               dsp0, dsp1, dsp2, dsp3, sem):
    c = lax.axis_index("c")
    s = lax.axis_index("s")
    wid = s * NC + c
    dsps = [dsp0, dsp1, dsp2, dsp3]

    def _z(i, carry):
        zbuf[pl.ds(i * 16, 16)] = jnp.zeros((16,), jnp.float32)
        return carry
    lax.fori_loop(0, RPT // 16, _z, 0)
    for hh in range(H):
        pltpu.sync_copy(zbuf, dsps[hh].at[pl.ds(s * RPT, RPT)])

    pltpu.sync_copy(asrc_h, asv)
    pltpu.sync_copy(adst_h, adv)
    base = wid * EA
    pltpu.sync_copy(srcp_h.at[pl.ds(base, EA)], srcb)
    pltpu.sync_copy(dst2_h.at[wid], dstb2)
    plsc.subcore_barrier()

    def _blk(b, carry):
        eb = base + b * 128
        for g in range(8):
            s16 = srcb[pl.ds(b * 128 + g * 16, 16)] * H
            d16 = dstb2[b, pl.ds(g * 16, 16)] * H
            eid = eb + g * 16 + _iota16()
            valid = eid < EN
            for hh in range(H):
                av = plsc.load_gather(asv, [s16 + hh])
                bv = plsc.load_gather(adv, [d16 + hh])
                al = av + bv
                al = jnp.maximum(al, 0.2 * al)
                ev = jnp.where(valid, jnp.exp(al), 0.0)
                exbuf[hh, pl.ds(b * 128 + g * 16, 16)] = ev
        for hh in range(H):
            pltpu.sync_copy(exbuf.at[hh, pl.ds(b * 128, 128)],
                            dsps[hh].at[dstb2.at[b]], add=True)
        return carry
    lax.fori_loop(0, BA, _blk, 0)

    for hh in range(H):
        pltpu.sync_copy(exbuf.at[hh], exh_h.at[hh, pl.ds(base, EA)])
    plsc.subcore_barrier()
    for hh in range(H):
        pltpu.sync_copy(dsps[hh].at[pl.ds(s * RPT, RPT)],
                        dpart_h.at[c, hh, pl.ds(s * RPT, RPT)])


def _sc_edges(a_src_flat, a_dst_flat, srcp, dst2):
    return pl.kernel(
        _edge_body,
        out_type=[
            jax.ShapeDtypeStruct((H, EP), jnp.float32),
            jax.ShapeDtypeStruct((NC, H, NP), jnp.float32),
        ],
        mesh=_mesh(),
        compiler_params=pltpu.CompilerParams(needs_layout_passes=False),
        scratch_types=[
            pltpu.VMEM((N * H,), jnp.float32),
            pltpu.VMEM((N * H,), jnp.float32),
            pltpu.VMEM((EA,), jnp.int32),
            pltpu.VMEM((BA, 128), jnp.int32),
            pltpu.VMEM((H, EA), jnp.float32),
            pltpu.VMEM((RPT,), jnp.float32),
            pltpu.VMEM_SHARED((NP,), jnp.float32),
            pltpu.VMEM_SHARED((NP,), jnp.float32),
            pltpu.VMEM_SHARED((NP,), jnp.float32),
            pltpu.VMEM_SHARED((NP,), jnp.float32),
            pltpu.SemaphoreType.DMA,
        ],
    )(a_src_flat, a_dst_flat, srcp, dst2)


def _agg_body(hflat_h, srcp_h, dst2_h, exh_h, dpart_h, bias_h,
              outp_h,
              sv_big, dv2, iv2, ex_big,
              rbf0, rbf1, rbf2, rows0, rows1, rows2,
              rinvf, bv, accsp,
              sem_g0, sem_g1, sem_g2,
              sem_s0, sem_s1, sem_s2):
    c = lax.axis_index("c")
    s = lax.axis_index("s")
    row0 = s * RPT
    rbf_b = (rbf0, rbf1, rbf2)
    rows_b = (rows0, rows1, rows2)
    sem_g = (sem_g0, sem_g1, sem_g2)
    sem_s = (sem_s0, sem_s1, sem_s2)

    for hh in range(H):
        pltpu.sync_copy(dpart_h.at[0, hh, pl.ds(row0, RPT)],
                        rinvf.at[pl.ds(hh * RPT, RPT)])
        pltpu.sync_copy(dpart_h.at[1, hh, pl.ds(row0, RPT)],
                        ex_big.at[pl.ds(0, RPT)])

        def _r(i, carry):
            a = rinvf[pl.ds(hh * RPT + i * 16, 16)]
            b = ex_big[pl.ds(i * 16, 16)]
            rinvf[pl.ds(hh * RPT + i * 16, 16)] = 1.0 / (a + b + 1e-16)
            return carry
        lax.fori_loop(0, RPT // 16, _r, 0)

    def _chunk(ch, carry0):
        cg = c * (NCH // NC) + ch
        head = cg >> 1
        fbase = cg * F

        def _z(r, carry):
            for k in range(8):
                rows0[r, pl.ds(k * 16, 16)] = jnp.zeros((16,), jnp.float32)
            return carry
        lax.fori_loop(0, BLK, _z, 0)

        def _zc(k, carry):
            pltpu.sync_copy(rows0, accsp.at[pl.ds(row0 + k * BLK, BLK)])
            return carry
        lax.fori_loop(0, RPT // BLK, _zc, 0)
        plsc.subcore_barrier()

        def _group(gi, carry):
            geb = s * EC + gi * GE
            gbb = s * (EC // BLK) + gi * GB
            pltpu.sync_copy(srcp_h.at[pl.ds(geb, GE)], sv_big)
            pltpu.sync_copy(dst2_h.at[pl.ds(gbb, GB)], dv2)
            pltpu.sync_copy(exh_h.at[head, pl.ds(geb, GE)], ex_big)
            for kb in range(GB):
                for j in range(BLK // 16):
                    iv2[kb, pl.ds(j * 16, 16)] = (
                        sv_big[pl.ds(kb * BLK + j * 16, 16)] + cg * N)

            def _gat(kb, gbuf):
                return pltpu.async_copy(hflat_h.at[iv2.at[kb]],
                                        rbf_b[gbuf], sem_g[gbuf])

            _gat(0, 0)
            _gat(1, 1)

            def _tri(sb, carry2):
                for q in range(3):
                    kb = sb * 3 + q
                    gbuf = q
                    pltpu.make_async_copy(hflat_h.at[iv2.at[kb]],
                                          rbf_b[gbuf], sem_g[gbuf]).wait()

                    @pl.when(kb >= 3)
                    def _():
                        pltpu.make_async_copy(rows_b[gbuf],
                                              accsp.at[dv2.at[kb - 3]],
                                              sem_s[gbuf]).wait()

                    @plsc.parallel_loop(0, BLK, step=1, unroll=4)
                    def _scale(j):
                        sc = _splat(ex_big, kb * BLK + j)
                        for k in range(4):
                            w = rbf_b[gbuf][j, pl.ds(k * 16, 16)]
                            ev = plsc.bitcast(jnp.left_shift(w, 16),
                                              jnp.float32)
                            od = plsc.bitcast(
                                jnp.bitwise_and(w, jnp.int32(-65536)),
                                jnp.float32)
                            rows_b[gbuf][j, pl.ds(k * 32, 16)] = ev * sc
                            rows_b[gbuf][j, pl.ds(k * 32 + 16, 16)] = od * sc

                    pltpu.async_copy(rows_b[gbuf], accsp.at[dv2.at[kb]],
                                     sem_s[gbuf], add=True)

                    @pl.when(kb + 2 < GB)
                    def _():
                        _gat(kb + 2, (q + 2) % 3)
                return carry2
            lax.fori_loop(0, GB // 3, _tri, 0)
            for kb in range(GB - 3, GB):
                pltpu.make_async_copy(rows_b[kb % 3], accsp.at[dv2.at[kb]],
                                      sem_s[kb % 3]).wait()
            return carry
        lax.fori_loop(0, NG, _group, 0)
        plsc.subcore_barrier()

        pltpu.sync_copy(bias_h.at[pl.ds(fbase, F)], bv)

        def _fink(k5, carry):
            r0 = row0 + k5 * BLK
            pltpu.sync_copy(accsp.at[pl.ds(r0, BLK)], rows0)

            @plsc.parallel_loop(0, BLK, step=1, unroll=4)
            def _fin(r):
                sc = _splat(rinvf, head * RPT + k5 * BLK + r)
                for k in range(8):
                    rows0[r, pl.ds(k * 16, 16)] = jnp.maximum(
                        rows0[r, pl.ds(k * 16, 16)] * sc
                        + bv[pl.ds(k * 16, 16)], 0.0)
            pltpu.sync_copy(rows0, outp_h.at[pl.ds(r0, BLK), pl.ds(fbase, F)])
            return carry
        lax.fori_loop(0, RPT // BLK, _fink, 0)
        plsc.subcore_barrier()
        return carry0
    lax.fori_loop(0, NCH // NC, _chunk, 0)


def _sc_aggregate(hflat, srcp, dst2, exh, dpart, bias):
    return pl.kernel(
        _agg_body,
        out_type=jax.ShapeDtypeStruct((NP, H * C), jnp.float32),
        mesh=_mesh(),
        compiler_params=pltpu.CompilerParams(needs_layout_passes=False,
                                             use_tc_tiling_on_sc=False),
        scratch_types=[
            pltpu.VMEM((GE,), jnp.int32),
            pltpu.VMEM((GB, BLK), jnp.int32),
            pltpu.VMEM((GB, BLK), jnp.int32),
            pltpu.VMEM((GE,), jnp.float32),
            pltpu.VMEM((BLK, F // 2), jnp.int32),
            pltpu.VMEM((BLK, F // 2), jnp.int32),
            pltpu.VMEM((BLK, F // 2), jnp.int32),
            pltpu.VMEM((BLK, F), jnp.float32),
            pltpu.VMEM((BLK, F), jnp.float32),
            pltpu.VMEM((BLK, F), jnp.float32),
            pltpu.VMEM((RPT * H,), jnp.float32),
            pltpu.VMEM((F,), jnp.float32),
            pltpu.VMEM_SHARED((NP, F), jnp.float32),
            pltpu.SemaphoreType.DMA,
            pltpu.SemaphoreType.DMA,
            pltpu.SemaphoreType.DMA,
            pltpu.SemaphoreType.DMA,
            pltpu.SemaphoreType.DMA,
            pltpu.SemaphoreType.DMA,
        ],
    )(hflat, srcp, dst2, exh, dpart, bias)


def kernel(x, edge_index, W, att_src, att_dst, bias):
    eye = jnp.eye(H, dtype=jnp.float32)
    Ms = (att_src.reshape(H, C)[:, :, None] * eye[:, None, :]).reshape(H * C, H)
    Md = (att_dst.reshape(H, C)[:, :, None] * eye[:, None, :]).reshape(H * C, H)

    loop = jnp.arange(N, dtype=edge_index.dtype)
    pad = jnp.arange(EP - EN, dtype=edge_index.dtype) % N
    srcp = jnp.concatenate([edge_index[0], loop, pad]).astype(jnp.int32)
    dstp = jnp.concatenate([edge_index[1], loop, pad]).astype(jnp.int32)

    hch, a_src, a_dst = _tc_matmul(x, W, Ms, Md)
    hflat = (hch.reshape(NCH, N, F // 32, 2, 16)
             .transpose(0, 1, 2, 4, 3).reshape(NCH * N, F // 2, 2))
    hflat = jax.lax.bitcast_convert_type(hflat, jnp.int32)

    exh, dpart = _sc_edges(a_src.reshape(N * H), a_dst.reshape(N * H),
                           srcp, dstp.reshape(NC * NS, BA, 128))
    outp = _sc_aggregate(hflat, srcp, dstp.reshape(EP // BLK, BLK),
                         exh, dpart, bias)
    return outp[:N]
